# baseline (speedup 1.0000x reference)
import jax
import jax.numpy as jnp
from jax.experimental import pallas as pl
from jax.experimental.pallas import tpu as pltpu

BM = 200


def _gcn_body(inp_ref, w_ref, adja_ref, adjb_ref, bias_ref, out_ref, inpb_ref):
    @pl.when(pl.program_id(0) == 0)
    def _cast_input():
        inpb_ref[...] = inp_ref[...].astype(jnp.bfloat16)

    w = w_ref[...].astype(jnp.bfloat16)

    ta = jnp.dot(
        adja_ref[0].astype(jnp.bfloat16),
        inpb_ref[...],
        preferred_element_type=jnp.float32,
    )
    out_ref[0] = jnp.dot(
        ta.astype(jnp.bfloat16), w, preferred_element_type=jnp.float32
    ) + bias_ref[...]

    tb = jnp.dot(
        adjb_ref[0].astype(jnp.bfloat16),
        inpb_ref[...],
        preferred_element_type=jnp.float32,
    )
    out_ref[1] = jnp.dot(
        tb.astype(jnp.bfloat16), w, preferred_element_type=jnp.float32
    ) + bias_ref[...]


@jax.jit
def kernel(input, adj, weight, bias):
    n, f_in = input.shape
    f_out = weight.shape[1]
    bias2d = bias.reshape(1, f_out)
    half = n // 2
    adj3 = adj.reshape(2, half, n)
    grid = (half // BM,)
    out3 = pl.pallas_call(
        _gcn_body,
        grid=grid,
        in_specs=[
            pl.BlockSpec((n, f_in), lambda i: (0, 0)),
            pl.BlockSpec((f_in, f_out), lambda i: (0, 0)),
            pl.BlockSpec((1, BM, n), lambda i: (0, i, 0)),
            pl.BlockSpec((1, BM, n), lambda i: (1, i, 0)),
            pl.BlockSpec((1, f_out), lambda i: (0, 0)),
        ],
        out_specs=pl.BlockSpec((2, BM, f_out), lambda i: (0, i, 0)),
        out_shape=jax.ShapeDtypeStruct((2, half, f_out), jnp.float32),
        scratch_shapes=[pltpu.VMEM((n, f_in), jnp.bfloat16)],
        compiler_params=pltpu.CompilerParams(
            dimension_semantics=("arbitrary",),
            vmem_limit_bytes=100 * 1024 * 1024,
        ),
    )(input, weight, adj3, adj3, bias2d)
    return out3.reshape(n, f_out)


# speedup vs baseline: 1.1299x; 1.1299x over previous
import jax
import jax.numpy as jnp
from jax.experimental import pallas as pl
from jax.experimental.pallas import tpu as pltpu

BM = 400


def _gcn_body(inp_ref, w_ref, adj_ref, bias_ref, out_ref, inpb_ref):
    @pl.when(pl.program_id(0) == 0)
    def _cast_input():
        inpb_ref[...] = inp_ref[...].astype(jnp.bfloat16)

    t = jnp.dot(
        adj_ref[...].astype(jnp.bfloat16),
        inpb_ref[...],
        preferred_element_type=jnp.float32,
    )
    acc = jnp.dot(
        t.astype(jnp.bfloat16),
        w_ref[...].astype(jnp.bfloat16),
        preferred_element_type=jnp.float32,
    )
    out_ref[...] = acc + bias_ref[...]


@jax.jit
def kernel(input, adj, weight, bias):
    n, f_in = input.shape
    f_out = weight.shape[1]
    bias2d = bias.reshape(1, f_out)
    grid = (pl.cdiv(n, BM),)
    out = pl.pallas_call(
        _gcn_body,
        grid=grid,
        in_specs=[
            pl.BlockSpec((n, f_in), lambda i: (0, 0)),
            pl.BlockSpec((f_in, f_out), lambda i: (0, 0)),
            pl.BlockSpec((BM, n), lambda i: (i, 0)),
            pl.BlockSpec((1, f_out), lambda i: (0, 0)),
        ],
        out_specs=pl.BlockSpec((BM, f_out), lambda i: (i, 0)),
        out_shape=jax.ShapeDtypeStruct((n, f_out), jnp.float32),
        scratch_shapes=[pltpu.VMEM((n, f_in), jnp.bfloat16)],
        compiler_params=pltpu.CompilerParams(
            dimension_semantics=("arbitrary",),
            vmem_limit_bytes=100 * 1024 * 1024,
        ),
    )(input, weight, adj, bias2d)
    return out
